# Initial kernel scaffold; baseline (speedup 1.0000x reference)
import jax
import jax.numpy as jnp
from jax.experimental import pallas as pl


def kernel(input_ids, embed_table, W_gnn, b_gnn, W_out, b_out):
    raise NotImplementedError("write your pallas kernel here")


# speedup vs baseline: 1.7296x; 1.7296x over previous
import functools

import jax
import jax.numpy as jnp
from jax import lax
from jax.experimental import pallas as pl
from jax.experimental.pallas import tpu as pltpu
from jax.experimental.pallas import tpu_sc as plsc

_HIDDEN = 128
_LANES = 16
_NC = 2
_NS = 16
_NW = _NC * _NS
_CHUNK = 128


def _sc_gather_sum(chunks_per_worker: int):
    mesh = plsc.VectorSubcoreMesh(core_axis_name="c", subcore_axis_name="s")
    n_acc = _HIDDEN // _LANES
    per_worker = chunks_per_worker * _CHUNK

    @functools.partial(
        pl.kernel,
        out_type=jax.ShapeDtypeStruct((_NW * _HIDDEN,), jnp.float32),
        mesh=mesh,
        scratch_types=[
            pltpu.VMEM((per_worker,), jnp.int32),
            pltpu.VMEM((_CHUNK, _HIDDEN), jnp.float32),
            pltpu.VMEM((_HIDDEN,), jnp.float32),
            pltpu.SemaphoreType.DMA,
        ],
    )
    def sc_body(ids_hbm, table_hbm, out_hbm, idx_v, rows_v, acc_v, sem):
        wid = lax.axis_index("s") * _NC + lax.axis_index("c")
        pltpu.sync_copy(ids_hbm.at[pl.ds(wid * per_worker, per_worker)], idx_v)

        def chunk_body(c, accs):
            idx_c = idx_v.at[pl.ds(c * _CHUNK, _CHUNK)]
            pltpu.async_copy(table_hbm.at[idx_c], rows_v, sem).wait()

            def row_body(r, a):
                return tuple(a[k] + rows_v[r, pl.ds(k * _LANES, _LANES)]
                             for k in range(n_acc))

            return lax.fori_loop(0, _CHUNK, row_body, accs)

        zero = jnp.zeros((_LANES,), jnp.float32)
        accs = lax.fori_loop(0, chunks_per_worker, chunk_body, (zero,) * n_acc)
        for k in range(n_acc):
            acc_v[pl.ds(k * _LANES, _LANES)] = accs[k]
        pltpu.sync_copy(acc_v, out_hbm.at[pl.ds(wid * _HIDDEN, _HIDDEN)])

    return sc_body


def _tc_finish_body(p_ref, e0_ref, wg_ref, bg_ref, wo_ref, bo_ref, o_ref,
                    *, pad_count, n_rows):
    s = jnp.sum(p_ref[...], axis=0, keepdims=True)
    s = s - jnp.float32(pad_count) * e0_ref[...]
    m = s * jnp.float32(1.0 / n_rows)
    h = jnp.dot(m, wg_ref[...], preferred_element_type=jnp.float32,
                precision=lax.Precision.HIGHEST) + bg_ref[...]
    o = jnp.dot(h, wo_ref[...], preferred_element_type=jnp.float32,
                precision=lax.Precision.HIGHEST) + bo_ref[...]
    o_ref[...] = o


def kernel(input_ids, embed_table, W_gnn, b_gnn, W_out, b_out):
    n = input_ids.shape[0]
    hidden = embed_table.shape[1]
    out_dim = W_out.shape[1]
    per_worker_ids = -(-n // (_NW * _CHUNK)) * _CHUNK
    padded = per_worker_ids * _NW
    pad = padded - n

    ids = input_ids.astype(jnp.int32)
    if pad:
        ids = jnp.concatenate([ids, jnp.zeros((pad,), jnp.int32)])

    partials = _sc_gather_sum(per_worker_ids // _CHUNK)(ids, embed_table)
    partials = partials.reshape(_NW, hidden)

    e0 = lax.slice(embed_table, (0, 0), (1, hidden))
    out = pl.pallas_call(
        functools.partial(_tc_finish_body, pad_count=pad, n_rows=n),
        out_shape=jax.ShapeDtypeStruct((1, out_dim), jnp.float32),
    )(partials, e0, W_gnn, b_gnn.reshape(1, hidden), W_out,
      b_out.reshape(1, out_dim))
    return out.reshape(out_dim)


# speedup vs baseline: 5.0627x; 2.9271x over previous
import functools

import jax
import jax.numpy as jnp
from jax import lax
from jax.experimental import pallas as pl
from jax.experimental.pallas import tpu as pltpu
from jax.experimental.pallas import tpu_sc as plsc

_LANES = 16
_NC = 2
_NS = 16
_NW = _NC * _NS
_IDS_UNROLL = 4
_VB = 1024


def _sc_histogram(per_worker: int, vpad: int):
    mesh = plsc.VectorSubcoreMesh(core_axis_name="c", subcore_axis_name="s")

    @functools.partial(
        pl.kernel,
        out_type=jax.ShapeDtypeStruct((_NW * vpad,), jnp.int32),
        mesh=mesh,
        compiler_params=pltpu.CompilerParams(needs_layout_passes=False),
        scratch_types=[
            pltpu.VMEM((per_worker,), jnp.int32),
            pltpu.VMEM((vpad,), jnp.int32),
        ],
    )
    def sc_body(ids_hbm, out_hbm, idx_v, hist_v):
        wid = lax.axis_index("s") * _NC + lax.axis_index("c")
        pltpu.sync_copy(ids_hbm.at[pl.ds(wid * per_worker, per_worker)], idx_v)

        zero = jnp.zeros((_LANES,), jnp.int32)

        def zero_body(i, _):
            for u in range(16):
                hist_v[pl.ds((i * 16 + u) * _LANES, _LANES)] = zero
            return 0

        lax.fori_loop(0, vpad // (16 * _LANES), zero_body, 0)

        ones = jnp.ones((_LANES,), jnp.int32)

        def hist_body(i, _):
            for u in range(_IDS_UNROLL):
                ids16 = idx_v[pl.ds((i * _IDS_UNROLL + u) * _LANES, _LANES)]
                plsc.addupdate_scatter(hist_v, [ids16], ones)
            return 0

        lax.fori_loop(0, per_worker // (_IDS_UNROLL * _LANES), hist_body, 0)
        pltpu.sync_copy(hist_v, out_hbm.at[pl.ds(wid * vpad, vpad)])

    return sc_body


def _tc_contract_body(h_ref, t_ref, e0_ref, wg_ref, bg_ref, wo_ref, bo_ref,
                      o_ref, acc_ref, *, vocab, n_rows, pad_count, n_blocks):
    k = pl.program_id(0)

    @pl.when(k == 0)
    def _init():
        acc_ref[...] = jnp.zeros_like(acc_ref)

    counts = jnp.sum(h_ref[...], axis=0, keepdims=True).astype(jnp.float32)
    row = lax.broadcasted_iota(jnp.int32, t_ref.shape, 0) + k * t_ref.shape[0]
    tb = jnp.where(row < vocab, t_ref[...], 0.0)
    acc_ref[...] += jnp.dot(counts, tb, preferred_element_type=jnp.float32,
                            precision=lax.Precision.HIGHEST)

    @pl.when(k == n_blocks - 1)
    def _finish():
        s = acc_ref[...] - jnp.float32(pad_count) * e0_ref[...]
        m = s * jnp.float32(1.0 / n_rows)
        h = jnp.dot(m, wg_ref[...], preferred_element_type=jnp.float32,
                    precision=lax.Precision.HIGHEST) + bg_ref[...]
        o = jnp.dot(h, wo_ref[...], preferred_element_type=jnp.float32,
                    precision=lax.Precision.HIGHEST) + bo_ref[...]
        o_ref[...] = o


def kernel(input_ids, embed_table, W_gnn, b_gnn, W_out, b_out):
    n = input_ids.shape[0]
    vocab, hidden = embed_table.shape
    out_dim = W_out.shape[1]
    ids_align = _IDS_UNROLL * _LANES
    per_worker = -(-n // (_NW * ids_align)) * ids_align
    pad = per_worker * _NW - n
    vpad = -(-vocab // _VB) * _VB
    n_blocks = vpad // _VB

    ids = input_ids.astype(jnp.int32)
    if pad:
        ids = jnp.concatenate([ids, jnp.zeros((pad,), jnp.int32)])

    hist = _sc_histogram(per_worker, vpad)(ids).reshape(_NW, vpad)

    e0 = lax.slice(embed_table, (0, 0), (1, hidden))
    out = pl.pallas_call(
        functools.partial(_tc_contract_body, vocab=vocab, n_rows=n,
                          pad_count=pad, n_blocks=n_blocks),
        grid=(n_blocks,),
        in_specs=[
            pl.BlockSpec((_NW, _VB), lambda k: (0, k)),
            pl.BlockSpec((_VB, hidden), lambda k: (k, 0)),
            pl.BlockSpec((1, hidden), lambda k: (0, 0)),
            pl.BlockSpec((hidden, hidden), lambda k: (0, 0)),
            pl.BlockSpec((1, hidden), lambda k: (0, 0)),
            pl.BlockSpec((hidden, out_dim), lambda k: (0, 0)),
            pl.BlockSpec((1, out_dim), lambda k: (0, 0)),
        ],
        out_specs=pl.BlockSpec((1, out_dim), lambda k: (0, 0)),
        out_shape=jax.ShapeDtypeStruct((1, out_dim), jnp.float32),
        scratch_shapes=[pltpu.VMEM((1, hidden), jnp.float32)],
    )(hist, embed_table, e0, W_gnn, b_gnn.reshape(1, hidden), W_out,
      b_out.reshape(1, out_dim))
    return out.reshape(out_dim)


# speedup vs baseline: 5.9332x; 1.1720x over previous
import functools

import jax
import jax.numpy as jnp
from jax import lax
from jax.experimental import pallas as pl
from jax.experimental.pallas import tpu as pltpu
from jax.experimental.pallas import tpu_sc as plsc

_LANES = 16
_NC = 2
_NS = 16
_NW = _NC * _NS
_IDS_UNROLL = 4
_VB = 2048


def _sc_histogram(per_worker: int, vpad: int):
    mesh = plsc.VectorSubcoreMesh(core_axis_name="c", subcore_axis_name="s")

    @functools.partial(
        pl.kernel,
        out_type=jax.ShapeDtypeStruct((_NW * vpad,), jnp.int32),
        mesh=mesh,
        compiler_params=pltpu.CompilerParams(needs_layout_passes=False),
        scratch_types=[
            pltpu.VMEM((per_worker,), jnp.int32),
            pltpu.VMEM((vpad,), jnp.int32),
        ],
    )
    def sc_body(ids_hbm, out_hbm, idx_v, hist_v):
        wid = lax.axis_index("s") * _NC + lax.axis_index("c")
        pltpu.sync_copy(ids_hbm.at[pl.ds(wid * per_worker, per_worker)], idx_v)

        zero = jnp.zeros((_LANES,), jnp.int32)

        def zero_body(i, _):
            for u in range(16):
                hist_v[pl.ds((i * 16 + u) * _LANES, _LANES)] = zero
            return 0

        lax.fori_loop(0, vpad // (16 * _LANES), zero_body, 0)

        ones = jnp.ones((_LANES,), jnp.int32)

        def hist_body(i, _):
            for u in range(_IDS_UNROLL):
                ids16 = idx_v[pl.ds((i * _IDS_UNROLL + u) * _LANES, _LANES)]
                plsc.addupdate_scatter(hist_v, [ids16], ones)
            return 0

        lax.fori_loop(0, per_worker // (_IDS_UNROLL * _LANES), hist_body, 0)
        pltpu.sync_copy(hist_v, out_hbm.at[pl.ds(wid * vpad, vpad)])

    return sc_body


def _tc_contract_body(h_ref, t_ref, e0_ref, wg_ref, bg_ref, wo_ref, bo_ref,
                      o_ref, acc_ref, *, vocab, n_rows, pad_count, n_blocks):
    k = pl.program_id(0)

    @pl.when(k == 0)
    def _init():
        acc_ref[...] = jnp.zeros_like(acc_ref)

    counts = jnp.sum(h_ref[...], axis=0, keepdims=True).astype(jnp.float32)
    blk = t_ref.shape[0]

    @pl.when(k < n_blocks - 1)
    def _full():
        acc_ref[...] += jnp.dot(counts, t_ref[...],
                                preferred_element_type=jnp.float32,
                                precision=lax.Precision.HIGHEST)

    @pl.when(k == n_blocks - 1)
    def _masked():
        row = lax.broadcasted_iota(jnp.int32, t_ref.shape, 0) + k * blk
        tb = jnp.where(row < vocab, t_ref[...], 0.0)
        acc_ref[...] += jnp.dot(counts, tb,
                                preferred_element_type=jnp.float32,
                                precision=lax.Precision.HIGHEST)

    @pl.when(k == n_blocks - 1)
    def _finish():
        s = acc_ref[...] - jnp.float32(pad_count) * e0_ref[...]
        m = s * jnp.float32(1.0 / n_rows)
        h = jnp.dot(m, wg_ref[...], preferred_element_type=jnp.float32,
                    precision=lax.Precision.HIGHEST) + bg_ref[...]
        o = jnp.dot(h, wo_ref[...], preferred_element_type=jnp.float32,
                    precision=lax.Precision.HIGHEST) + bo_ref[...]
        o_ref[...] = o


def kernel(input_ids, embed_table, W_gnn, b_gnn, W_out, b_out):
    n = input_ids.shape[0]
    vocab, hidden = embed_table.shape
    out_dim = W_out.shape[1]
    ids_align = _IDS_UNROLL * _LANES
    per_worker = -(-n // (_NW * ids_align)) * ids_align
    pad = per_worker * _NW - n
    vpad = -(-vocab // _VB) * _VB
    n_blocks = vpad // _VB

    ids = input_ids.astype(jnp.int32)
    if pad:
        ids = jnp.concatenate([ids, jnp.zeros((pad,), jnp.int32)])

    hist = _sc_histogram(per_worker, vpad)(ids).reshape(_NW, vpad)

    e0 = lax.slice(embed_table, (0, 0), (1, hidden))
    out = pl.pallas_call(
        functools.partial(_tc_contract_body, vocab=vocab, n_rows=n,
                          pad_count=pad, n_blocks=n_blocks),
        grid=(n_blocks,),
        in_specs=[
            pl.BlockSpec((_NW, _VB), lambda k: (0, k)),
            pl.BlockSpec((_VB, hidden), lambda k: (k, 0)),
            pl.BlockSpec((1, hidden), lambda k: (0, 0)),
            pl.BlockSpec((hidden, hidden), lambda k: (0, 0)),
            pl.BlockSpec((1, hidden), lambda k: (0, 0)),
            pl.BlockSpec((hidden, out_dim), lambda k: (0, 0)),
            pl.BlockSpec((1, out_dim), lambda k: (0, 0)),
        ],
        out_specs=pl.BlockSpec((1, out_dim), lambda k: (0, 0)),
        out_shape=jax.ShapeDtypeStruct((1, out_dim), jnp.float32),
        scratch_shapes=[pltpu.VMEM((1, hidden), jnp.float32)],
    )(hist, embed_table, e0, W_gnn, b_gnn.reshape(1, hidden), W_out,
      b_out.reshape(1, out_dim))
    return out.reshape(out_dim)


# speedup vs baseline: 6.2633x; 1.0556x over previous
import functools

import jax
import jax.numpy as jnp
from jax import lax
from jax.experimental import pallas as pl
from jax.experimental.pallas import tpu as pltpu
from jax.experimental.pallas import tpu_sc as plsc

_LANES = 16
_NC = 2
_NS = 16
_NW = _NC * _NS
_IDS_UNROLL = 4
_VB = 2048


def _sc_histogram(per_worker: int, vpad: int):
    mesh = plsc.VectorSubcoreMesh(core_axis_name="c", subcore_axis_name="s")

    @functools.partial(
        pl.kernel,
        out_type=jax.ShapeDtypeStruct((_NW * vpad,), jnp.int32),
        mesh=mesh,
        compiler_params=pltpu.CompilerParams(needs_layout_passes=False),
        scratch_types=[
            pltpu.VMEM((per_worker,), jnp.int32),
            pltpu.VMEM((vpad,), jnp.int32),
        ],
    )
    def sc_body(ids_hbm, out_hbm, idx_v, hist_v):
        wid = lax.axis_index("s") * _NC + lax.axis_index("c")
        pltpu.sync_copy(ids_hbm.at[pl.ds(wid * per_worker, per_worker)], idx_v)

        zero = jnp.zeros((_LANES,), jnp.int32)

        def zero_body(i, _):
            for u in range(16):
                hist_v[pl.ds((i * 16 + u) * _LANES, _LANES)] = zero
            return 0

        lax.fori_loop(0, vpad // (16 * _LANES), zero_body, 0)

        ones = jnp.ones((_LANES,), jnp.int32)

        def hist_body(i, _):
            for u in range(_IDS_UNROLL):
                ids16 = idx_v[pl.ds((i * _IDS_UNROLL + u) * _LANES, _LANES)]
                plsc.addupdate_scatter(hist_v, [ids16], ones)
            return 0

        lax.fori_loop(0, per_worker // (_IDS_UNROLL * _LANES), hist_body, 0)
        pltpu.sync_copy(hist_v, out_hbm.at[pl.ds(wid * vpad, vpad)])

    return sc_body


def _tc_contract_body(h_ref, t_ref, e0_ref, wg_ref, bg_ref, wo_ref, bo_ref,
                      o_ref, acc_ref, *, vocab, n_rows, pad_count, n_blocks):
    k = pl.program_id(0)

    @pl.when(k == 0)
    def _init():
        acc_ref[...] = jnp.zeros_like(acc_ref)

    counts = jnp.sum(h_ref[...], axis=0, keepdims=True).astype(jnp.float32)
    blk = t_ref.shape[0]

    @pl.when(k < n_blocks - 1)
    def _full():
        acc_ref[...] += jnp.dot(counts, t_ref[...],
                                preferred_element_type=jnp.float32)

    @pl.when(k == n_blocks - 1)
    def _masked():
        row = lax.broadcasted_iota(jnp.int32, t_ref.shape, 0) + k * blk
        tb = jnp.where(row < vocab, t_ref[...], 0.0)
        acc_ref[...] += jnp.dot(counts, tb,
                                preferred_element_type=jnp.float32)

    @pl.when(k == n_blocks - 1)
    def _finish():
        s = acc_ref[...] - jnp.float32(pad_count) * e0_ref[...]
        m = s * jnp.float32(1.0 / n_rows)
        h = jnp.dot(m, wg_ref[...], preferred_element_type=jnp.float32,
                    precision=lax.Precision.HIGHEST) + bg_ref[...]
        o = jnp.dot(h, wo_ref[...], preferred_element_type=jnp.float32,
                    precision=lax.Precision.HIGHEST) + bo_ref[...]
        o_ref[...] = o


def kernel(input_ids, embed_table, W_gnn, b_gnn, W_out, b_out):
    n = input_ids.shape[0]
    vocab, hidden = embed_table.shape
    out_dim = W_out.shape[1]
    ids_align = _IDS_UNROLL * _LANES
    per_worker = -(-n // (_NW * ids_align)) * ids_align
    pad = per_worker * _NW - n
    vpad = -(-vocab // _VB) * _VB
    n_blocks = vpad // _VB

    ids = input_ids.astype(jnp.int32)
    if pad:
        ids = jnp.concatenate([ids, jnp.zeros((pad,), jnp.int32)])

    hist = _sc_histogram(per_worker, vpad)(ids).reshape(_NW, vpad)

    e0 = lax.slice(embed_table, (0, 0), (1, hidden))
    out = pl.pallas_call(
        functools.partial(_tc_contract_body, vocab=vocab, n_rows=n,
                          pad_count=pad, n_blocks=n_blocks),
        grid=(n_blocks,),
        in_specs=[
            pl.BlockSpec((_NW, _VB), lambda k: (0, k)),
            pl.BlockSpec((_VB, hidden), lambda k: (k, 0)),
            pl.BlockSpec((1, hidden), lambda k: (0, 0)),
            pl.BlockSpec((hidden, hidden), lambda k: (0, 0)),
            pl.BlockSpec((1, hidden), lambda k: (0, 0)),
            pl.BlockSpec((hidden, out_dim), lambda k: (0, 0)),
            pl.BlockSpec((1, out_dim), lambda k: (0, 0)),
        ],
        out_specs=pl.BlockSpec((1, out_dim), lambda k: (0, 0)),
        out_shape=jax.ShapeDtypeStruct((1, out_dim), jnp.float32),
        scratch_shapes=[pltpu.VMEM((1, hidden), jnp.float32)],
    )(hist, embed_table, e0, W_gnn, b_gnn.reshape(1, hidden), W_out,
      b_out.reshape(1, out_dim))
    return out.reshape(out_dim)


# speedup vs baseline: 6.9275x; 1.1061x over previous
import functools

import jax
import jax.numpy as jnp
from jax import lax
from jax.experimental import pallas as pl
from jax.experimental.pallas import tpu as pltpu
from jax.experimental.pallas import tpu_sc as plsc

_LANES = 16
_NC = 2
_NS = 16
_NW = _NC * _NS
_IDS_UNROLL = 4
_VB = 2048


def _sc_histogram(n: int, wlen: int, vpad: int):
    mesh = plsc.VectorSubcoreMesh(core_axis_name="c", subcore_axis_name="s")
    vrows = vpad // 128

    @functools.partial(
        pl.kernel,
        out_type=jax.ShapeDtypeStruct((_NW, vrows, 128), jnp.int32),
        mesh=mesh,
        compiler_params=pltpu.CompilerParams(needs_layout_passes=False),
        scratch_types=[
            pltpu.VMEM((wlen,), jnp.int32),
            pltpu.VMEM((vrows, 128), jnp.int32),
        ],
    )
    def sc_body(ids_hbm, out_hbm, idx_v, hist_v):
        wid = lax.axis_index("s") * _NC + lax.axis_index("c")
        start = (wid * n) // _NW
        end = ((wid + 1) * n) // _NW
        astart = jnp.minimum((start // 8) * 8, n - wlen)
        pltpu.sync_copy(ids_hbm.at[pl.ds(astart, wlen)], idx_v)

        zero = jnp.zeros((_LANES,), jnp.int32)

        def zero_body(r, _):
            for u in range(128 // _LANES):
                hist_v[r, pl.ds(u * _LANES, _LANES)] = zero
            return 0

        lax.fori_loop(0, vrows, zero_body, 0)

        ones = jnp.ones((_LANES,), jnp.int32)
        lane = lax.iota(jnp.int32, _LANES)

        def hist_body(i, _):
            for u in range(_IDS_UNROLL):
                g = i * _IDS_UNROLL + u
                ids16 = idx_v[pl.ds(g * _LANES, _LANES)]
                pos = astart + g * _LANES + lane
                m = jnp.logical_and(pos >= start, pos < end)
                row16 = lax.shift_right_logical(ids16, 7)
                col16 = jnp.bitwise_and(ids16, 127)
                plsc.addupdate_scatter(hist_v, [row16, col16], ones, mask=m)
            return 0

        lax.fori_loop(0, wlen // (_IDS_UNROLL * _LANES), hist_body, 0)
        pltpu.sync_copy(hist_v, out_hbm.at[wid])

    return sc_body


def _tc_contract_body(h_ref, t_ref, wg_ref, bg_ref, wo_ref, bo_ref,
                      o_ref, acc_ref, *, vocab, n_rows, n_blocks):
    k = pl.program_id(0)

    @pl.when(k == 0)
    def _init():
        acc_ref[...] = jnp.zeros_like(acc_ref)

    counts = jnp.sum(h_ref[...], axis=0)
    c = counts.astype(jnp.float32).reshape(1, -1)
    blk = t_ref.shape[0]

    @pl.when(k < n_blocks - 1)
    def _full():
        acc_ref[...] += jnp.dot(c, t_ref[...],
                                preferred_element_type=jnp.float32)

    @pl.when(k == n_blocks - 1)
    def _masked():
        row = lax.broadcasted_iota(jnp.int32, t_ref.shape, 0) + k * blk
        tb = jnp.where(row < vocab, t_ref[...], 0.0)
        acc_ref[...] += jnp.dot(c, tb, preferred_element_type=jnp.float32)

    @pl.when(k == n_blocks - 1)
    def _finish():
        m = acc_ref[...] * jnp.float32(1.0 / n_rows)
        h = jnp.dot(m, wg_ref[...], preferred_element_type=jnp.float32,
                    precision=lax.Precision.HIGHEST) + bg_ref[...]
        o = jnp.dot(h, wo_ref[...], preferred_element_type=jnp.float32,
                    precision=lax.Precision.HIGHEST) + bo_ref[...]
        o_ref[...] = o


def kernel(input_ids, embed_table, W_gnn, b_gnn, W_out, b_out):
    n = input_ids.shape[0]
    vocab, hidden = embed_table.shape
    out_dim = W_out.shape[1]
    max_span = -(-n // _NW)
    wlen = -(-(max_span + 7) // _LANES) * _LANES
    vpad = -(-vocab // _VB) * _VB
    n_blocks = vpad // _VB

    ids = input_ids.astype(jnp.int32)
    hist = _sc_histogram(n, wlen, vpad)(ids)

    out = pl.pallas_call(
        functools.partial(_tc_contract_body, vocab=vocab, n_rows=n,
                          n_blocks=n_blocks),
        grid=(n_blocks,),
        in_specs=[
            pl.BlockSpec((_NW, _VB // 128, 128), lambda k: (0, k, 0)),
            pl.BlockSpec((_VB, hidden), lambda k: (k, 0)),
            pl.BlockSpec((hidden, hidden), lambda k: (0, 0)),
            pl.BlockSpec((1, hidden), lambda k: (0, 0)),
            pl.BlockSpec((hidden, out_dim), lambda k: (0, 0)),
            pl.BlockSpec((1, out_dim), lambda k: (0, 0)),
        ],
        out_specs=pl.BlockSpec((1, out_dim), lambda k: (0, 0)),
        out_shape=jax.ShapeDtypeStruct((1, out_dim), jnp.float32),
        scratch_shapes=[pltpu.VMEM((1, hidden), jnp.float32)],
    )(hist, embed_table, W_gnn, b_gnn.reshape(1, hidden), W_out,
      b_out.reshape(1, out_dim))
    return out.reshape(out_dim)


# speedup vs baseline: 7.1383x; 1.0304x over previous
import functools

import jax
import jax.numpy as jnp
from jax import lax
from jax.experimental import pallas as pl
from jax.experimental.pallas import tpu as pltpu
from jax.experimental.pallas import tpu_sc as plsc

_LANES = 16
_NC = 2
_NS = 16
_NW = _NC * _NS
_IDS_UNROLL = 4
_VB = 2048


def _sc_histogram(n: int, wlen: int, vpad: int):
    mesh = plsc.VectorSubcoreMesh(core_axis_name="c", subcore_axis_name="s")
    vrows = vpad // 128

    @functools.partial(
        pl.kernel,
        out_type=jax.ShapeDtypeStruct((_NW, vrows, 128), jnp.int32),
        mesh=mesh,
        compiler_params=pltpu.CompilerParams(needs_layout_passes=False),
        scratch_types=[
            pltpu.VMEM((wlen,), jnp.int32),
            pltpu.VMEM((vrows, 128), jnp.int32),
        ],
    )
    def sc_body(ids_hbm, out_hbm, idx_v, hist_v):
        wid = lax.axis_index("s") * _NC + lax.axis_index("c")
        start = (wid * n) // _NW
        end = ((wid + 1) * n) // _NW
        astart = jnp.minimum((start // 8) * 8, n - wlen)
        pltpu.sync_copy(ids_hbm.at[pl.ds(astart, wlen)], idx_v)

        zero = jnp.zeros((_LANES,), jnp.int32)

        def zero_body(r, _):
            for u in range(128 // _LANES):
                hist_v[r, pl.ds(u * _LANES, _LANES)] = zero
            return 0

        lax.fori_loop(0, vrows, zero_body, 0)

        ones = jnp.ones((_LANES,), jnp.int32)
        lane = lax.iota(jnp.int32, _LANES)

        def scatter16(ids16, m=None):
            row16 = lax.shift_right_logical(ids16, 7)
            col16 = jnp.bitwise_and(ids16, 127)
            plsc.addupdate_scatter(hist_v, [row16, col16], ones, mask=m)

        n_groups = wlen // _LANES
        g_lo = (start - astart + _LANES - 1) // _LANES
        g_hi = (end - astart) // _LANES

        def edge_body(g, _):
            pos = astart + g * _LANES + lane
            m = jnp.logical_and(pos >= start, pos < end)
            scatter16(idx_v[pl.ds(g * _LANES, _LANES)], m)
            return 0

        lax.fori_loop(0, g_lo, edge_body, 0)
        n_mid = (g_hi - g_lo) // _IDS_UNROLL

        def mid_strided(i, _):
            for u in range(_IDS_UNROLL):
                g = g_lo + i * _IDS_UNROLL + u
                scatter16(idx_v[pl.ds(g * _LANES, _LANES)])
            return 0

        lax.fori_loop(0, n_mid, mid_strided, 0)
        lax.fori_loop(g_lo + n_mid * _IDS_UNROLL, n_groups, edge_body, 0)
        pltpu.sync_copy(hist_v, out_hbm.at[wid])

    return sc_body


def _tc_contract_body(h_ref, t_ref, wg_ref, bg_ref, wo_ref, bo_ref,
                      o_ref, acc_ref, *, vocab, n_rows, n_blocks):
    k = pl.program_id(0)

    @pl.when(k == 0)
    def _init():
        acc_ref[...] = jnp.zeros_like(acc_ref)

    counts = jnp.sum(h_ref[...], axis=0)
    c = counts.astype(jnp.bfloat16).reshape(1, -1)
    blk = t_ref.shape[0]

    @pl.when(k < n_blocks - 1)
    def _full():
        acc_ref[...] += jnp.dot(c, t_ref[...].astype(jnp.bfloat16),
                                preferred_element_type=jnp.float32)

    @pl.when(k == n_blocks - 1)
    def _masked():
        row = lax.broadcasted_iota(jnp.int32, t_ref.shape, 0) + k * blk
        tb = jnp.where(row < vocab, t_ref[...], 0.0).astype(jnp.bfloat16)
        acc_ref[...] += jnp.dot(c, tb, preferred_element_type=jnp.float32)

    @pl.when(k == n_blocks - 1)
    def _finish():
        m = acc_ref[...] * jnp.float32(1.0 / n_rows)
        h = jnp.dot(m, wg_ref[...], preferred_element_type=jnp.float32,
                    precision=lax.Precision.HIGHEST) + bg_ref[...]
        o = jnp.dot(h, wo_ref[...], preferred_element_type=jnp.float32,
                    precision=lax.Precision.HIGHEST) + bo_ref[...]
        o_ref[...] = o.reshape(o_ref.shape)


def kernel(input_ids, embed_table, W_gnn, b_gnn, W_out, b_out):
    n = input_ids.shape[0]
    vocab, hidden = embed_table.shape
    out_dim = W_out.shape[1]
    max_span = -(-n // _NW)
    wlen = -(-(max_span + 7) // _LANES) * _LANES
    vpad = -(-vocab // _VB) * _VB
    n_blocks = vpad // _VB

    ids = input_ids.astype(jnp.int32)
    hist = _sc_histogram(n, wlen, vpad)(ids)

    out = pl.pallas_call(
        functools.partial(_tc_contract_body, vocab=vocab, n_rows=n,
                          n_blocks=n_blocks),
        grid=(n_blocks,),
        in_specs=[
            pl.BlockSpec((_NW, _VB // 128, 128), lambda k: (0, k, 0)),
            pl.BlockSpec((_VB, hidden), lambda k: (k, 0)),
            pl.BlockSpec((hidden, hidden), lambda k: (0, 0)),
            pl.BlockSpec((1, hidden), lambda k: (0, 0)),
            pl.BlockSpec((hidden, out_dim), lambda k: (0, 0)),
            pl.BlockSpec((1, out_dim), lambda k: (0, 0)),
        ],
        out_specs=pl.BlockSpec((out_dim,), lambda k: (0,)),
        out_shape=jax.ShapeDtypeStruct((out_dim,), jnp.float32),
        scratch_shapes=[pltpu.VMEM((1, hidden), jnp.float32)],
    )(hist, embed_table, W_gnn, b_gnn.reshape(1, hidden), W_out,
      b_out.reshape(1, out_dim))
    return out
